# baseline (speedup 1.0000x reference)
import functools

import jax
import jax.numpy as jnp
from jax import lax
from jax.experimental import pallas as pl
from jax.experimental.pallas import tpu as pltpu

N = 10000
NFEAT = 128
NHID = 64
NCLASS = 32
ALPHA = 0.2
BM = 200


def _prep_kernel(x_ref, w_ref, asrc_ref, adstT_ref,
                 wh_ref, a_ref, b_ref, fdst_ref, fd2_ref, meanwh_ref):
    wh = jnp.dot(x_ref[...], w_ref[...], preferred_element_type=jnp.float32)
    wh_ref[...] = wh
    fsrc = jnp.dot(wh, asrc_ref[...], preferred_element_type=jnp.float32)
    fdst = lax.dot_general(adstT_ref[...], wh, (((1,), (1,)), ((), ())),
                           preferred_element_type=jnp.float32)
    fdst_ref[...] = fdst
    fd2_ref[...] = ALPHA * fdst
    maxd = jnp.max(fdst)
    s = fsrc + maxd
    m = jnp.maximum(s, ALPHA * s)
    a_ref[...] = fsrc - m
    b_ref[...] = ALPHA * fsrc - m
    meanwh_ref[...] = jnp.mean(wh, axis=0, keepdims=True)


def _prep(x, w, a, out_dim):
    asrc = a[:out_dim]
    adstT = a[out_dim:].T
    n, _ = x.shape
    return pl.pallas_call(
        _prep_kernel,
        out_shape=(
            jax.ShapeDtypeStruct((n, out_dim), jnp.float32),
            jax.ShapeDtypeStruct((n, 1), jnp.float32),
            jax.ShapeDtypeStruct((n, 1), jnp.float32),
            jax.ShapeDtypeStruct((1, n), jnp.float32),
            jax.ShapeDtypeStruct((1, n), jnp.float32),
            jax.ShapeDtypeStruct((1, out_dim), jnp.float32),
        ),
    )(x, w, asrc, adstT)


def _layer1_kernel(adj_ref, wh_ref, a_ref, b_ref, fdst_ref, fd2_ref, meanwh_ref,
                   out_ref, hz_ref):
    t = jnp.maximum(a_ref[...] + fdst_ref[...], b_ref[...] + fd2_ref[...])
    p = jnp.exp(t)
    unmasked = adj_ref[...] > 0
    p = jnp.where(unmasked, p, 0.0)
    hz_ref[...] = jnp.sum(jnp.where(unmasked, 0.0, 1.0), axis=1, keepdims=True)
    denom = jnp.sum(p, axis=1, keepdims=True)
    o = jnp.dot(p, wh_ref[...], preferred_element_type=jnp.float32)
    o = jnp.where(denom > 0, o / denom, meanwh_ref[...])
    o = jnp.where(o > 0, o, jnp.exp(o) - 1.0)
    out_ref[...] = o


def _layer1(adj, prepped, out_dim):
    n = adj.shape[0]
    return pl.pallas_call(
        _layer1_kernel,
        grid=(n // BM,),
        in_specs=[
            pl.BlockSpec((BM, n), lambda i: (i, 0)),
            pl.BlockSpec((n, out_dim), lambda i: (0, 0)),
            pl.BlockSpec((BM, 1), lambda i: (i, 0)),
            pl.BlockSpec((BM, 1), lambda i: (i, 0)),
            pl.BlockSpec((1, n), lambda i: (0, 0)),
            pl.BlockSpec((1, n), lambda i: (0, 0)),
            pl.BlockSpec((1, out_dim), lambda i: (0, 0)),
        ],
        out_specs=(
            pl.BlockSpec((BM, out_dim), lambda i: (i, 0)),
            pl.BlockSpec((BM, 1), lambda i: (i, 0)),
        ),
        out_shape=(
            jax.ShapeDtypeStruct((n, out_dim), jnp.float32),
            jax.ShapeDtypeStruct((n, 1), jnp.float32),
        ),
    )(adj, *prepped)


def _layer2_kernel(zflags_ref, adj_hbm, wh_ref, a_ref, b_ref, fdst_ref,
                   fd2_ref, meanwh_ref, out_ref, p_scr, adj_scr, sem):
    i = pl.program_id(0)
    t = jnp.maximum(a_ref[...] + fdst_ref[...], b_ref[...] + fd2_ref[...])
    p_scr[...] = jnp.exp(t)

    @pl.when(zflags_ref[i] != 0)
    def _apply_mask():
        copy = pltpu.make_async_copy(
            adj_hbm.at[pl.ds(i * BM, BM), :], adj_scr, sem)
        copy.start()
        copy.wait()
        p_scr[...] = jnp.where(adj_scr[...] > 0, p_scr[...], 0.0)

    p = p_scr[...]
    denom = jnp.sum(p, axis=1, keepdims=True)
    o = jnp.dot(p, wh_ref[...], preferred_element_type=jnp.float32)
    out_ref[...] = jnp.where(denom > 0, o / denom, meanwh_ref[...])


def _layer2(zflags, adj, prepped, out_dim):
    n = adj.shape[0]
    grid_spec = pltpu.PrefetchScalarGridSpec(
        num_scalar_prefetch=1,
        grid=(n // BM,),
        in_specs=[
            pl.BlockSpec(memory_space=pltpu.MemorySpace.HBM),
            pl.BlockSpec((n, out_dim), lambda i, zf: (0, 0)),
            pl.BlockSpec((BM, 1), lambda i, zf: (i, 0)),
            pl.BlockSpec((BM, 1), lambda i, zf: (i, 0)),
            pl.BlockSpec((1, n), lambda i, zf: (0, 0)),
            pl.BlockSpec((1, n), lambda i, zf: (0, 0)),
            pl.BlockSpec((1, out_dim), lambda i, zf: (0, 0)),
        ],
        out_specs=pl.BlockSpec((BM, out_dim), lambda i, zf: (i, 0)),
        scratch_shapes=[
            pltpu.VMEM((BM, n), jnp.float32),
            pltpu.VMEM((BM, n), jnp.float32),
            pltpu.SemaphoreType.DMA,
        ],
    )
    return pl.pallas_call(
        _layer2_kernel,
        grid_spec=grid_spec,
        out_shape=jax.ShapeDtypeStruct((n, out_dim), jnp.float32),
    )(zflags, adj, *prepped)


@jax.jit
def kernel(x, adj, W1, a1, W2, a2):
    prepped1 = _prep(x, W1, a1, NHID)
    h, hz = _layer1(adj, prepped1, NHID)
    zflags = (hz.reshape(N // BM, BM).sum(axis=1) > 0).astype(jnp.int32)
    prepped2 = _prep(h, W2, a2, NCLASS)
    return _layer2(zflags, adj, prepped2, NCLASS)
